# baseline (speedup 1.0000x reference)
import functools

import jax
import jax.numpy as jnp
from jax import lax
from jax.experimental import pallas as pl

_BETA = 0.25


def _prep_body(cb_ref, cbn_ref, hi_ref, lo_ref, lo2_ref, *, num_layers):
    for layer in range(num_layers):
        cb = cb_ref[layer]
        norm = jnp.sqrt(jnp.sum(cb * cb, axis=1, keepdims=True))
        cbn = cb / jnp.maximum(norm, 1e-12)
        cbn_ref[layer] = cbn
        hi = cb.astype(jnp.bfloat16)
        hi_ref[layer] = hi
        r1 = cb - hi.astype(jnp.float32)
        lo = r1.astype(jnp.bfloat16)
        lo_ref[layer] = lo
        lo2_ref[layer] = (r1 - lo.astype(jnp.float32)).astype(jnp.bfloat16)


def _body(x_ref, cbn_ref, hi_ref, lo_ref, lo2_ref, ids_ref, dec_ref,
          loss_ref, *, num_layers, block_n, k, chains):
    i = pl.program_id(0)
    h = block_n // chains
    iota = lax.broadcasted_iota(
        jnp.int32, (k, h), 0).astype(jnp.float32)
    kf = jnp.float32(k)
    resids = [x_ref[c * h:(c + 1) * h, :] for c in range(chains)]
    accs = [jnp.float32(0.0) for _ in range(chains)]
    for layer in range(num_layers):
        cbn = cbn_ref[layer]
        hi = hi_ref[layer]
        lo = lo_ref[layer]
        lo2 = lo2_ref[layer]
        b2 = jnp.sum(cbn * cbn, axis=1, keepdims=True)
        for c in range(chains):
            resid = resids[c]
            a2 = jnp.sum(resid * resid, axis=1, keepdims=True)
            a2t = jnp.transpose(a2, (1, 0))
            s = lax.dot_general(
                cbn, resid, (((1,), (1,)), ((), ())),
                precision=lax.Precision.DEFAULT,
                preferred_element_type=jnp.float32)
            d2 = jnp.sqrt(jnp.maximum((a2t + b2) - 2.0 * s, 0.0))
            minv = jnp.min(d2, axis=0, keepdims=True)
            idxf = jnp.min(jnp.where(d2 == minv, iota, kf), axis=0)
            ids_ref[layer, c * h:(c + 1) * h] = idxf.astype(jnp.int32)
            onehot = (iota == idxf[None, :]).astype(jnp.bfloat16)
            q = lax.dot_general(
                onehot, hi, (((0,), (0,)), ((), ())),
                preferred_element_type=jnp.float32)
            q = q + lax.dot_general(
                onehot, lo, (((0,), (0,)), ((), ())),
                preferred_element_type=jnp.float32)
            q = q + lax.dot_general(
                onehot, lo2, (((0,), (0,)), ((), ())),
                preferred_element_type=jnp.float32)
            ste = resid + (q - resid)
            resids[c] = resid - ste
            accs[c] = accs[c] + jnp.sum(resids[c] * resids[c])
    for c in range(chains):
        dec_ref[c * h:(c + 1) * h, :] = (
            x_ref[c * h:(c + 1) * h, :] - resids[c])
    acc = sum(accs)

    @pl.when(i == 0)
    def _():
        loss_ref[...] = jnp.zeros((1, 1), jnp.float32)

    loss_ref[...] = loss_ref[...] + acc


def kernel(x, codebooks):
    n, d = x.shape
    num_layers, k, _ = codebooks.shape
    block_n = min(n, 1024)
    assert n % block_n == 0

    cbn, cb_hi, cb_lo, cb_lo2 = pl.pallas_call(
        functools.partial(_prep_body, num_layers=num_layers),
        out_shape=[
            jax.ShapeDtypeStruct((num_layers, k, d), jnp.float32),
            jax.ShapeDtypeStruct((num_layers, k, d), jnp.bfloat16),
            jax.ShapeDtypeStruct((num_layers, k, d), jnp.bfloat16),
            jax.ShapeDtypeStruct((num_layers, k, d), jnp.bfloat16),
        ],
    )(codebooks)

    whole = pl.BlockSpec((num_layers, k, d), lambda i: (0, 0, 0))
    ids, dec, loss = pl.pallas_call(
        functools.partial(_body, num_layers=num_layers, block_n=block_n, k=k,
                          chains=2 if block_n % 2 == 0 else 1),
        grid=(n // block_n,),
        in_specs=[
            pl.BlockSpec((block_n, d), lambda i: (i, 0)),
            whole, whole, whole, whole,
        ],
        out_specs=[
            pl.BlockSpec((num_layers, block_n), lambda i: (0, i)),
            pl.BlockSpec((block_n, d), lambda i: (i, 0)),
            pl.BlockSpec((1, 1), lambda i: (0, 0)),
        ],
        out_shape=[
            jax.ShapeDtypeStruct((num_layers, n), jnp.int32),
            jax.ShapeDtypeStruct((n, d), jnp.float32),
            jax.ShapeDtypeStruct((1, 1), jnp.float32),
        ],
    )(x, cbn, cb_hi, cb_lo, cb_lo2)

    scale = jnp.float32((1.0 + _BETA) / (n * d))
    return (ids.T, dec, (loss[0, 0] * scale).astype(jnp.float32))


# speedup vs baseline: 1.5059x; 1.5059x over previous
import functools

import jax
import jax.numpy as jnp
from jax import lax
from jax.experimental import pallas as pl

_BETA = 0.25


def _prep_body(cb_ref, cbn_ref, hi_ref, lo_ref, *, num_layers):
    for layer in range(num_layers):
        cb = cb_ref[layer]
        norm = jnp.sqrt(jnp.sum(cb * cb, axis=1, keepdims=True))
        cbn = cb / jnp.maximum(norm, 1e-12)
        cbn_ref[layer] = cbn
        hi = cb.astype(jnp.bfloat16)
        hi_ref[layer] = hi
        lo_ref[layer] = (cb - hi.astype(jnp.float32)).astype(jnp.bfloat16)


def _body(x_ref, cbn_ref, hi_ref, lo_ref, ids_ref, dec_ref,
          loss_ref, *, num_layers, block_n, k, chains):
    i = pl.program_id(0)
    h = block_n // chains
    iota = lax.broadcasted_iota(
        jnp.int32, (k, h), 0).astype(jnp.float32)
    kf = jnp.float32(k)
    groups = 8
    gw = k // groups
    gwf = jnp.float32(gw)
    gf = jnp.float32(groups)
    iota1 = lax.broadcasted_iota(
        jnp.int32, (groups, gw, h), 1).astype(jnp.float32)
    giota = lax.broadcasted_iota(
        jnp.int32, (groups, h), 0).astype(jnp.float32)
    resids = [x_ref[c * h:(c + 1) * h, :] for c in range(chains)]
    accs = [jnp.float32(0.0) for _ in range(chains)]
    for layer in range(num_layers):
        cbn = cbn_ref[layer]
        hi = hi_ref[layer]
        lo = lo_ref[layer]
        b2 = jnp.sum(cbn * cbn, axis=1, keepdims=True)
        for c in range(chains):
            resid = resids[c]
            a2 = jnp.sum(resid * resid, axis=1, keepdims=True)
            a2t = jnp.transpose(a2, (1, 0))
            s = lax.dot_general(
                cbn, resid, (((1,), (1,)), ((), ())),
                precision=lax.Precision.DEFAULT,
                preferred_element_type=jnp.float32)
            d2 = (a2t + b2) - 2.0 * s
            d2g = d2.reshape(groups, k // groups, h)
            pm = jnp.min(d2g, axis=1)
            idx1 = jnp.min(
                jnp.where(d2g == pm[:, None, :], iota1, gwf), axis=1)
            dsq = jnp.sqrt(pm)
            msq = jnp.min(dsq, axis=0, keepdims=True)
            gidx = jnp.min(jnp.where(dsq == msq, giota, gf), axis=0)
            gid = giota * gwf + idx1
            idxf = jnp.min(
                jnp.where(giota == gidx[None, :], gid, kf), axis=0)
            ids_ref[layer, c * h:(c + 1) * h] = idxf.astype(jnp.int32)
            onehot = (iota == idxf[None, :]).astype(jnp.bfloat16)
            q = lax.dot_general(
                onehot, hi, (((0,), (0,)), ((), ())),
                preferred_element_type=jnp.float32)
            q = q + lax.dot_general(
                onehot, lo, (((0,), (0,)), ((), ())),
                preferred_element_type=jnp.float32)
            ste = resid + (q - resid)
            resids[c] = resid - ste
            accs[c] = accs[c] + jnp.sum(resids[c] * resids[c])
    for c in range(chains):
        dec_ref[c * h:(c + 1) * h, :] = (
            x_ref[c * h:(c + 1) * h, :] - resids[c])
    acc = sum(accs)

    @pl.when(i == 0)
    def _():
        loss_ref[...] = jnp.zeros((1, 1), jnp.float32)

    loss_ref[...] = loss_ref[...] + acc


def kernel(x, codebooks):
    n, d = x.shape
    num_layers, k, _ = codebooks.shape
    block_n = min(n, 1024)
    assert n % block_n == 0

    cbn, cb_hi, cb_lo = pl.pallas_call(
        functools.partial(_prep_body, num_layers=num_layers),
        out_shape=[
            jax.ShapeDtypeStruct((num_layers, k, d), jnp.float32),
            jax.ShapeDtypeStruct((num_layers, k, d), jnp.bfloat16),
            jax.ShapeDtypeStruct((num_layers, k, d), jnp.bfloat16),
        ],
    )(codebooks)

    whole = pl.BlockSpec((num_layers, k, d), lambda i: (0, 0, 0))
    ids, dec, loss = pl.pallas_call(
        functools.partial(_body, num_layers=num_layers, block_n=block_n, k=k,
                          chains=2 if block_n % 2 == 0 else 1),
        grid=(n // block_n,),
        in_specs=[
            pl.BlockSpec((block_n, d), lambda i: (i, 0)),
            whole, whole, whole,
        ],
        out_specs=[
            pl.BlockSpec((num_layers, block_n), lambda i: (0, i)),
            pl.BlockSpec((block_n, d), lambda i: (i, 0)),
            pl.BlockSpec((1, 1), lambda i: (0, 0)),
        ],
        out_shape=[
            jax.ShapeDtypeStruct((num_layers, n), jnp.int32),
            jax.ShapeDtypeStruct((n, d), jnp.float32),
            jax.ShapeDtypeStruct((1, 1), jnp.float32),
        ],
    )(x, cbn, cb_hi, cb_lo)

    scale = jnp.float32((1.0 + _BETA) / (n * d))
    return (ids.T, dec, (loss[0, 0] * scale).astype(jnp.float32))
